# Initial kernel scaffold; baseline (speedup 1.0000x reference)
import jax
import jax.numpy as jnp
from jax.experimental import pallas as pl


def kernel(features, edge_index, edge_weight, W1, b1, W2, b2):
    raise NotImplementedError("write your pallas kernel here")


# speedup vs baseline: 4.1197x; 4.1197x over previous
import functools

import jax
import jax.numpy as jnp
from jax import lax
from jax.experimental import pallas as pl
from jax.experimental.pallas import tpu as pltpu
from jax.experimental.pallas import tpu_sc as plsc

N_NODES = 10000
N_EDGES = 320000
LANES = 16
NC = 2
NS = 16
NW = NC * NS
K_EDGES = 128


def _sc_spmm(h, src, dst, w, d_feat):
    n_chunks = N_EDGES // K_EDGES
    chunks_lo = n_chunks // NW
    n_extra = n_chunks - chunks_lo * NW
    row_stride = 624
    row_width = N_NODES - row_stride * (NS - 1)
    mesh = plsc.VectorSubcoreMesh(core_axis_name="c", subcore_axis_name="s",
                                  num_cores=NC, num_subcores=NS)

    @functools.partial(
        pl.kernel,
        out_type=jax.ShapeDtypeStruct((NC, N_NODES, d_feat), jnp.float32),
        mesh=mesh,
        scratch_types=[
            pltpu.VMEM((K_EDGES,), jnp.int32),
            pltpu.VMEM((K_EDGES,), jnp.int32),
            pltpu.VMEM((K_EDGES,), jnp.float32),
            pltpu.VMEM((K_EDGES, d_feat), jnp.float32),
            pltpu.VMEM_SHARED((N_NODES, d_feat), jnp.float32),
            pltpu.SemaphoreType.DMA,
        ],
        compiler_params=pltpu.CompilerParams(use_tc_tiling_on_sc=False),
    )
    def spmm(h_hbm, src_hbm, dst_hbm, w_hbm, zero_hbm, out_hbm,
             src_v, dst_v, w_v, rows_v, acc_sh, sem):
        cid = lax.axis_index("c")
        sid = lax.axis_index("s")
        wid = sid * NC + cid

        row0 = sid * row_stride
        pltpu.sync_copy(zero_hbm.at[pl.ds(row0, row_width)],
                        acc_sh.at[pl.ds(row0, row_width)])
        plsc.subcore_barrier()

        n_mine = chunks_lo + jnp.where(wid < n_extra, 1, 0)

        def chunk_body(i, carry):
            g = wid + i * NW
            base = g * K_EDGES
            pltpu.sync_copy(src_hbm.at[pl.ds(base, K_EDGES)], src_v)
            pltpu.sync_copy(w_hbm.at[pl.ds(base, K_EDGES)], w_v)
            pltpu.sync_copy(dst_hbm.at[pl.ds(base, K_EDGES)], dst_v)
            pltpu.async_copy(h_hbm.at[src_v], rows_v, sem).wait()

            def scale_group(g2, carry2):
                w16 = w_v[pl.ds(g2 * LANES, LANES)]
                for e in range(LANES):
                    idx = jnp.full((LANES,), e, dtype=jnp.int32)
                    wb = jnp.take(w16, idx)
                    row = g2 * LANES + e
                    for j in range(d_feat // LANES):
                        sl = pl.ds(j * LANES, LANES)
                        rows_v[row, sl] = rows_v[row, sl] * wb
                return carry2

            lax.fori_loop(0, K_EDGES // LANES, scale_group, 0, unroll=False)

            pltpu.sync_copy(rows_v, acc_sh.at[dst_v], add=True)
            return carry

        lax.fori_loop(0, n_mine, chunk_body, 0, unroll=False)

        plsc.subcore_barrier()
        pltpu.sync_copy(acc_sh.at[pl.ds(row0, row_width)],
                        out_hbm.at[cid].at[pl.ds(row0, row_width)])

    zeros = jnp.zeros((N_NODES, d_feat), jnp.float32)
    return spmm(h, src, dst, w, zeros)


def _tc_matmul(x, w):
    n, d_in = x.shape
    d_out = w.shape[1]
    br = 2000

    def mm(x_ref, w_ref, o_ref):
        o_ref[...] = jnp.dot(x_ref[...], w_ref[...],
                             preferred_element_type=jnp.float32)

    return pl.pallas_call(
        mm,
        out_shape=jax.ShapeDtypeStruct((n, d_out), jnp.float32),
        grid=(n // br,),
        in_specs=[
            pl.BlockSpec((br, d_in), lambda i: (i, 0)),
            pl.BlockSpec((d_in, d_out), lambda i: (0, 0)),
        ],
        out_specs=pl.BlockSpec((br, d_out), lambda i: (i, 0)),
    )(x, w)


def _tc_combine_relu_matmul(parts, b, w):
    _, n, d_in = parts.shape
    d_out = w.shape[1]
    br = 2000

    def body(p_ref, b_ref, w_ref, o_ref):
        z = jax.nn.relu(p_ref[0] + p_ref[1] + b_ref[...])
        o_ref[...] = jnp.dot(z, w_ref[...], preferred_element_type=jnp.float32)

    return pl.pallas_call(
        body,
        out_shape=jax.ShapeDtypeStruct((n, d_out), jnp.float32),
        grid=(n // br,),
        in_specs=[
            pl.BlockSpec((2, br, d_in), lambda i: (0, i, 0)),
            pl.BlockSpec((1, d_in), lambda i: (0, 0)),
            pl.BlockSpec((d_in, d_out), lambda i: (0, 0)),
        ],
        out_specs=pl.BlockSpec((br, d_out), lambda i: (i, 0)),
    )(parts, b.reshape(1, -1), w)


def _tc_combine_bias(parts, b):
    _, n, d = parts.shape
    br = 2000

    def body(p_ref, b_ref, o_ref):
        o_ref[...] = p_ref[0] + p_ref[1] + b_ref[...]

    return pl.pallas_call(
        body,
        out_shape=jax.ShapeDtypeStruct((n, d), jnp.float32),
        grid=(n // br,),
        in_specs=[
            pl.BlockSpec((2, br, d), lambda i: (0, i, 0)),
            pl.BlockSpec((1, d), lambda i: (0, 0)),
        ],
        out_specs=pl.BlockSpec((br, d), lambda i: (i, 0)),
    )(parts, b.reshape(1, -1))


def kernel(features, edge_index, edge_weight, W1, b1, W2, b2):
    src = edge_index[0].astype(jnp.int32)
    dst = edge_index[1].astype(jnp.int32)

    h1 = _tc_matmul(features, W1)
    p1 = _sc_spmm(h1, src, dst, edge_weight, W1.shape[1])
    h2 = _tc_combine_relu_matmul(p1, b1, W2)
    p2 = _sc_spmm(h2, src, dst, edge_weight, W2.shape[1])
    return _tc_combine_bias(p2, b2)
